# baseline (speedup 1.0000x reference)
import jax
import jax.numpy as jnp
from jax.experimental import pallas as pl
from jax.experimental.pallas import tpu as pltpu

_BLOCK = 512


def _mul_kernel(sp_ref, mask_ref, data_ref, out_ref):
    del sp_ref
    mask = mask_ref[0].astype(jnp.float32)
    out_ref[...] = data_ref[...] * mask


def kernel(data, selected_param, region_map):
    batch, size = data.shape
    sp = jnp.asarray(selected_param, jnp.int32).reshape((1,))
    rm3 = region_map.reshape(region_map.shape[0], 1, size)
    return pl.pallas_call(
        _mul_kernel,
        grid_spec=pltpu.PrefetchScalarGridSpec(
            num_scalar_prefetch=1,
            grid=(batch // _BLOCK,),
            in_specs=[
                pl.BlockSpec((1, 1, size), lambda i, sp: (sp[0], 0, 0)),
                pl.BlockSpec((_BLOCK, size), lambda i, sp: (i, 0)),
            ],
            out_specs=pl.BlockSpec((_BLOCK, size), lambda i, sp: (i, 0)),
        ),
        out_shape=jax.ShapeDtypeStruct((batch, size), jnp.float32),
        compiler_params=pltpu.CompilerParams(
            dimension_semantics=("arbitrary",)),
    )(sp, rm3, data)
